# baseline (speedup 1.0000x reference)
import functools

import numpy as np
import jax
import jax.numpy as jnp
from jax import lax
from jax.experimental import pallas as pl
from jax.experimental.pallas import tpu as pltpu
from jax.experimental.pallas import tpu_sc as plsc

_VOCAB = 1000000
_EMBED = 64
_SEQ = 200
_BATCH = 4096
_N = _BATCH * _SEQ

_NW = 32
_JB = _BATCH // 128
_UNITS = _SEQ * _JB
_PW = _UNITS // _NW
_GROUPS = _PW // 8
_L = 16
_DT = _EMBED // 8


def _make_pe():
    pos = np.arange(_SEQ, dtype=np.float32)[:, None]
    div = np.exp(np.arange(0, _EMBED, 2, dtype=np.float32)
                 * -(np.log(10000.0) / _EMBED))
    pe = np.zeros((_SEQ, _EMBED), np.float32)
    pe[:, 0::2] = np.sin(pos * div)
    pe[:, 1::2] = np.cos(pos * div)
    return pe


_PE = _make_pe()


def _sc_embed(seq_lin, tab2, pe):
    mesh = plsc.VectorSubcoreMesh(core_axis_name="c", subcore_axis_name="s")

    @functools.partial(
        pl.kernel,
        mesh=mesh,
        out_type=jax.ShapeDtypeStruct((_SEQ, _DT, _JB, 8, 128), jnp.float32),
        scratch_types=[
            pltpu.VMEM((8, 128), jnp.int32),
            pltpu.VMEM((8, 128), jnp.int32),
            pltpu.VMEM((2, 128, 128), jnp.float32),
            pltpu.VMEM((64, 128), jnp.float32),
            pltpu.VMEM((_SEQ, _EMBED), jnp.float32),
            pltpu.SemaphoreType.DMA,
            pltpu.SemaphoreType.DMA,
        ],
        compiler_params=pltpu.CompilerParams(use_tc_tiling_on_sc=False,
                                             needs_layout_passes=False),
    )
    def k(seq_hbm, tab_hbm, pe_hbm, out_hbm, idx_v, half_v, g2, st, pe_v,
          gsem, wsem):
        wid = lax.axis_index("s") * 2 + lax.axis_index("c")
        base_r = wid * _PW
        pltpu.sync_copy(pe_hbm, pe_v)
        tok_vecs = [lax.iota(jnp.int32, _L) + (h * _L) for h in range(8)]

        def group_body(g, carry):
            gr = base_r + g * 8
            gr8 = pl.multiple_of(gr, 8)
            pltpu.sync_copy(seq_hbm.at[pl.ds(gr8, 8)], idx_v)
            for u in range(8):
                for h in range(8):
                    sl = pl.ds(h * _L, _L)
                    half_v[u, sl] = lax.shift_right_logical(idx_v[u, sl], 1)

            def unit(u, prev_started):
                r = gr + u
                s = r // _JB
                j = lax.rem(r, _JB)
                cp = pltpu.async_copy(tab_hbm.at[half_v.at[u]],
                                      g2.at[u % 2], gsem)
                cp.wait()
                par = [
                    lax.shift_left(
                        lax.bitwise_and(idx_v[u, pl.ds(h * _L, _L)], 1), 6)
                    for h in range(8)
                ]
                s_splat = jnp.full((_L,), s, jnp.int32)

                def d_body(d, carry2):
                    pe_val = plsc.load_gather(
                        pe_v, [s_splat, jnp.full((_L,), d, jnp.int32)])
                    for h in range(8):
                        col = par[h] + d
                        v = plsc.load_gather(g2.at[u % 2],
                                             [tok_vecs[h], col])
                        st[d, pl.ds(h * _L, _L)] = v + pe_val
                    return carry2

                lax.fori_loop(0, _EMBED, d_body, 0)
                wcps = [
                    pltpu.async_copy(st.at[pl.ds(t * 8, 8)],
                                     out_hbm.at[s, t, j], wsem)
                    for t in range(_DT)
                ]
                for wc in wcps:
                    wc.wait()
                return prev_started

            for u in range(8):
                unit(u, 0)
            return carry

        lax.fori_loop(0, _GROUPS, group_body, 0)

    return k(seq_lin, tab2, pe)


@jax.jit
def kernel(sequence, table):
    seq_lin = jnp.transpose(sequence).reshape(_UNITS, 128).astype(jnp.int32)
    tab2 = jnp.reshape(table, (_VOCAB // 2, 128))
    pe = jnp.asarray(_PE)
    lin5 = _sc_embed(seq_lin, tab2, pe)
    return lin5.transpose(2, 4, 0, 1, 3).reshape(_BATCH, _SEQ, _EMBED)


# speedup vs baseline: 1.1508x; 1.1508x over previous
import functools

import numpy as np
import jax
import jax.numpy as jnp
from jax import lax
from jax.experimental import pallas as pl
from jax.experimental.pallas import tpu as pltpu
from jax.experimental.pallas import tpu_sc as plsc

_VOCAB = 1000000
_EMBED = 64
_SEQ = 200
_BATCH = 4096
_N = _BATCH * _SEQ

_NW = 32
_JB = _BATCH // 128
_UNITS = _SEQ * _JB
_PW = _UNITS // _NW
_GROUPS = _PW // 8
_L = 16
_DT = _EMBED // 8


def _make_pe():
    pos = np.arange(_SEQ, dtype=np.float32)[:, None]
    div = np.exp(np.arange(0, _EMBED, 2, dtype=np.float32)
                 * -(np.log(10000.0) / _EMBED))
    pe = np.zeros((_SEQ, _EMBED), np.float32)
    pe[:, 0::2] = np.sin(pos * div)
    pe[:, 1::2] = np.cos(pos * div)
    return pe


_PE = _make_pe()


def _sc_embed(seq_lin, table, pe):
    mesh = plsc.VectorSubcoreMesh(core_axis_name="c", subcore_axis_name="s")

    @functools.partial(
        pl.kernel,
        mesh=mesh,
        out_type=jax.ShapeDtypeStruct((_SEQ, _DT, _JB, 8, 128), jnp.float32),
        scratch_types=[
            pltpu.VMEM((8, 128), jnp.int32),
            pltpu.VMEM((2, 128, _EMBED), jnp.float32),
            pltpu.VMEM((2, _EMBED, 128), jnp.float32),
            pltpu.VMEM((_SEQ, _EMBED), jnp.float32),
            pltpu.SemaphoreType.DMA,
            pltpu.SemaphoreType.DMA,
        ],
        compiler_params=pltpu.CompilerParams(use_tc_tiling_on_sc=False,
                                             needs_layout_passes=False),
    )
    def k(seq_hbm, tab_hbm, pe_hbm, out_hbm, idx_v, g, st, pe_v, gsem, wsem):
        wid = lax.axis_index("s") * 2 + lax.axis_index("c")
        base_r = wid * _PW
        pltpu.sync_copy(pe_hbm, pe_v)
        tok_vecs = [lax.iota(jnp.int32, _L) + (h * _L) for h in range(8)]

        def gather(u):
            return pltpu.async_copy(tab_hbm.at[idx_v.at[u]], g.at[u % 2],
                                    gsem)

        def unit_compute(u, r):
            s = r // _JB
            j = lax.rem(r, _JB)
            s_splat = jnp.full((_L,), s, jnp.int32)

            def d_body(d, carry2):
                d_splat = jnp.full((_L,), d, jnp.int32)
                pe_val = plsc.load_gather(pe_v, [s_splat, d_splat])
                for h in range(8):
                    v = plsc.load_gather(g.at[u % 2], [tok_vecs[h], d_splat])
                    st[u % 2, d, pl.ds(h * _L, _L)] = v + pe_val
                return carry2

            lax.fori_loop(0, _EMBED, d_body, 0, unroll=4)
            return s, j

        def write(u, s, j):
            return [
                pltpu.async_copy(st.at[u % 2, pl.ds(t * 8, 8)],
                                 out_hbm.at[s, t, j], wsem)
                for t in range(_DT)
            ]

        def group_body(gi, carry):
            gr = base_r + gi * 8
            gr8 = pl.multiple_of(gr, 8)
            pltpu.sync_copy(seq_hbm.at[pl.ds(gr8, 8)], idx_v)
            cps = {0: gather(0)}
            wcs = {}
            for u in range(8):
                if u + 1 < 8:
                    cps[u + 1] = gather(u + 1)
                cps[u].wait()
                if u - 2 in wcs:
                    for wc in wcs.pop(u - 2):
                        wc.wait()
                s, j = unit_compute(u, gr + u)
                wcs[u] = write(u, s, j)
            for ws in wcs.values():
                for wc in ws:
                    wc.wait()
            return carry

        lax.fori_loop(0, _GROUPS, group_body, 0)

    return k(seq_lin, table, pe)


@jax.jit
def kernel(sequence, table):
    seq_lin = jnp.transpose(sequence).reshape(_UNITS, 128).astype(jnp.int32)
    pe = jnp.asarray(_PE)
    lin5 = _sc_embed(seq_lin, table, pe)
    return lin5.transpose(2, 4, 0, 1, 3).reshape(_BATCH, _SEQ, _EMBED)


# speedup vs baseline: 1.3885x; 1.2066x over previous
import functools

import numpy as np
import jax
import jax.numpy as jnp
from jax import lax
from jax.experimental import pallas as pl
from jax.experimental.pallas import tpu as pltpu
from jax.experimental.pallas import tpu_sc as plsc

_VOCAB = 1000000
_EMBED = 64
_SEQ = 200
_BATCH = 4096
_N = _BATCH * _SEQ

_NW = 32
_JB = _BATCH // 128
_UNITS = _SEQ * _JB
_PW = _UNITS // _NW
_GROUPS = _PW // 8
_L = 16
_DT = _EMBED // 8


def _make_pe():
    pos = np.arange(_SEQ, dtype=np.float32)[:, None]
    div = np.exp(np.arange(0, _EMBED, 2, dtype=np.float32)
                 * -(np.log(10000.0) / _EMBED))
    pe = np.zeros((_SEQ, _EMBED), np.float32)
    pe[:, 0::2] = np.sin(pos * div)
    pe[:, 1::2] = np.cos(pos * div)
    return pe


_PE = _make_pe()


def _sc_embed(seq_lin, table, pe):
    mesh = plsc.VectorSubcoreMesh(core_axis_name="c", subcore_axis_name="s")

    @functools.partial(
        pl.kernel,
        mesh=mesh,
        out_type=jax.ShapeDtypeStruct((_SEQ, _DT, _JB, 8, 128), jnp.float32),
        scratch_types=[
            pltpu.VMEM((8, 128), jnp.int32),
            pltpu.VMEM((8, 128, _EMBED), jnp.float32),
            pltpu.VMEM((2, _EMBED, 128), jnp.float32),
            pltpu.VMEM((_SEQ, _EMBED), jnp.float32),
            pltpu.SemaphoreType.DMA,
            pltpu.SemaphoreType.DMA,
        ],
        compiler_params=pltpu.CompilerParams(use_tc_tiling_on_sc=False,
                                             needs_layout_passes=False),
    )
    def k(seq_hbm, tab_hbm, pe_hbm, out_hbm, idx_v, g, st, pe_v, gsem, wsem):
        wid = lax.axis_index("s") * 2 + lax.axis_index("c")
        base_r = wid * _PW
        pltpu.sync_copy(pe_hbm, pe_v)
        tok_vecs = [lax.iota(jnp.int32, _L) + (h * _L) for h in range(8)]

        def unit_compute(u, r):
            s = r // _JB
            j = lax.rem(r, _JB)
            s_splat = jnp.full((_L,), s, jnp.int32)

            def d_body(d, carry2):
                d_splat = jnp.full((_L,), d, jnp.int32)
                pe_val = plsc.load_gather(pe_v, [s_splat, d_splat])
                vs = [plsc.load_gather(g.at[u], [tok_vecs[h], d_splat])
                      for h in range(8)]
                vs = [v + pe_val for v in vs]
                for h in range(8):
                    st[u % 2, d, pl.ds(h * _L, _L)] = vs[h]
                return carry2

            lax.fori_loop(0, _EMBED, d_body, 0, unroll=2)
            return s, j

        def write(u, s, j):
            return [
                pltpu.async_copy(st.at[u % 2, pl.ds(t * 8, 8)],
                                 out_hbm.at[s, t, j], wsem)
                for t in range(_DT)
            ]

        def group_body(gi, carry):
            gr = base_r + gi * 8
            gr8 = pl.multiple_of(gr, 8)
            pltpu.sync_copy(seq_hbm.at[pl.ds(gr8, 8)], idx_v)
            cps = [pltpu.async_copy(tab_hbm.at[idx_v.at[u]], g.at[u], gsem)
                   for u in range(8)]
            wcs = {}
            for u in range(8):
                cps[u].wait()
                if u - 2 in wcs:
                    for wc in wcs.pop(u - 2):
                        wc.wait()
                s, j = unit_compute(u, gr + u)
                wcs[u] = write(u, s, j)
            for ws in wcs.values():
                for wc in ws:
                    wc.wait()
            return carry

        lax.fori_loop(0, _GROUPS, group_body, 0)

    return k(seq_lin, table, pe)


@jax.jit
def kernel(sequence, table):
    seq_lin = jnp.transpose(sequence).reshape(_UNITS, 128).astype(jnp.int32)
    pe = jnp.asarray(_PE)
    lin5 = _sc_embed(seq_lin, table, pe)
    return lin5.transpose(2, 4, 0, 1, 3).reshape(_BATCH, _SEQ, _EMBED)


# speedup vs baseline: 1.3954x; 1.0050x over previous
import functools

import numpy as np
import jax
import jax.numpy as jnp
from jax import lax
from jax.experimental import pallas as pl
from jax.experimental.pallas import tpu as pltpu
from jax.experimental.pallas import tpu_sc as plsc

_VOCAB = 1000000
_EMBED = 64
_SEQ = 200
_BATCH = 4096
_N = _BATCH * _SEQ

_NW = 32
_JB = _BATCH // 128
_UNITS = _SEQ * _JB
_PW = _UNITS // _NW
_GROUPS = _PW // 8
_L = 16
_DT = _EMBED // 8


def _make_pe():
    pos = np.arange(_SEQ, dtype=np.float32)[:, None]
    div = np.exp(np.arange(0, _EMBED, 2, dtype=np.float32)
                 * -(np.log(10000.0) / _EMBED))
    pe = np.zeros((_SEQ, _EMBED), np.float32)
    pe[:, 0::2] = np.sin(pos * div)
    pe[:, 1::2] = np.cos(pos * div)
    return pe


_PE = _make_pe()


def _sc_embed(seq_lin, table, pe):
    mesh = plsc.VectorSubcoreMesh(core_axis_name="c", subcore_axis_name="s")

    @functools.partial(
        pl.kernel,
        mesh=mesh,
        out_type=jax.ShapeDtypeStruct((_SEQ, _DT, _JB, 8, 128), jnp.float32),
        scratch_types=[
            pltpu.VMEM((8, 128), jnp.int32),
            pltpu.VMEM((8, 128, _EMBED), jnp.float32),
            pltpu.VMEM((2, _DT, 8, 128), jnp.float32),
            pltpu.VMEM((_SEQ, _EMBED), jnp.float32),
            pltpu.SemaphoreType.DMA,
            pltpu.SemaphoreType.DMA,
        ],
        compiler_params=pltpu.CompilerParams(use_tc_tiling_on_sc=False,
                                             needs_layout_passes=False),
    )
    def k(seq_hbm, tab_hbm, pe_hbm, out_hbm, idx_v, g, st, pe_v, gsem, wsem):
        wid = lax.axis_index("s") * 2 + lax.axis_index("c")
        base_r = wid * _PW
        pltpu.sync_copy(pe_hbm, pe_v)
        tok_vecs = [lax.iota(jnp.int32, _L) + (h * _L) for h in range(8)]

        def unit_compute(u, r):
            s = r // _JB
            j = lax.rem(r, _JB)
            s_splat = jnp.full((_L,), s, jnp.int32)

            def d_body(d, carry2):
                d_splat = jnp.full((_L,), d, jnp.int32)
                pe_val = plsc.load_gather(pe_v, [s_splat, d_splat])
                vs = [plsc.load_gather(g.at[u], [tok_vecs[h], d_splat])
                      for h in range(8)]
                vs = [v + pe_val for v in vs]
                for h in range(8):
                    st[u % 2, d // 8, lax.rem(d, 8), pl.ds(h * _L, _L)] = vs[h]
                return carry2

            lax.fori_loop(0, _EMBED, d_body, 0, unroll=2)
            return s, j

        def write(u, s, j):
            return [pltpu.async_copy(st.at[u % 2], out_hbm.at[s, :, j], wsem)]

        def group_body(gi, carry):
            gr = base_r + gi * 8
            gr8 = pl.multiple_of(gr, 8)
            pltpu.sync_copy(seq_hbm.at[pl.ds(gr8, 8)], idx_v)
            cps = [pltpu.async_copy(tab_hbm.at[idx_v.at[u]], g.at[u], gsem)
                   for u in range(8)]
            wcs = {}
            for u in range(8):
                cps[u].wait()
                if u - 2 in wcs:
                    for wc in wcs.pop(u - 2):
                        wc.wait()
                s, j = unit_compute(u, gr + u)
                wcs[u] = write(u, s, j)
            for ws in wcs.values():
                for wc in ws:
                    wc.wait()
            return carry

        lax.fori_loop(0, _GROUPS, group_body, 0)

    return k(seq_lin, table, pe)


@jax.jit
def kernel(sequence, table):
    seq_lin = jnp.transpose(sequence).reshape(_UNITS, 128).astype(jnp.int32)
    pe = jnp.asarray(_PE)
    lin5 = _sc_embed(seq_lin, table, pe)
    return lin5.transpose(2, 4, 0, 1, 3).reshape(_BATCH, _SEQ, _EMBED)
